# baseline (speedup 1.0000x reference)
import functools

import jax
import jax.numpy as jnp
from jax import lax
from jax.experimental import pallas as pl
from jax.experimental.pallas import tpu as pltpu
from jax.experimental.pallas import tpu_sc as plsc

N = 10000
E = 320000
D = 128

NC = 2
NS = 16
NW = NC * NS
EP = 327680
EPW = EP // NW
ACCR = N + 8
TROW = 624
TAIL = N - NS * TROW
TAIL0 = NS * TROW
CW = 8


def _sc_body(with_count, o, k, nch, *refs):
    if with_count:
        (table, ei, zeros_o, zeros_c, out, cnt_out,
         src_idx, dst_idx, rows0, rows1, ones_v, acc, cnt_acc,
         sem0, sem1) = refs
    else:
        (table, ei, zeros_o, out,
         src_idx, dst_idx, rows0, rows1, acc, sem0, sem1) = refs
    c = lax.axis_index("c")
    s = lax.axis_index("s")
    wid = c * NS + s

    pltpu.sync_copy(ei.at[0, wid], src_idx)
    pltpu.sync_copy(ei.at[1, wid], dst_idx)

    row0 = s * TROW
    pltpu.sync_copy(zeros_o.at[pl.ds(0, TROW), :],
                    acc.at[pl.ds(row0, TROW), :])
    if with_count:
        pltpu.sync_copy(zeros_c.at[pl.ds(0, TROW), :],
                        cnt_acc.at[pl.ds(row0, TROW), :])

        def _onerow(i, _):
            ones_v[i, :] = jnp.ones((CW,), jnp.float32)
            return _
        lax.fori_loop(0, k, _onerow, None)

    @pl.when(s == 0)
    def _zero_tail():
        pltpu.sync_copy(zeros_o.at[pl.ds(TROW, TAIL), :],
                        acc.at[pl.ds(TAIL0, TAIL), :])
        if with_count:
            pltpu.sync_copy(zeros_c.at[pl.ds(TROW, TAIL), :],
                            cnt_acc.at[pl.ds(TAIL0, TAIL), :])
    plsc.subcore_barrier()

    def _consume(j, buf, sem):
        pltpu.make_async_copy(table.at[src_idx.at[j]], buf, sem).wait()
        pltpu.sync_copy(buf, acc.at[dst_idx.at[j]], add=True)
        if with_count:
            pltpu.sync_copy(ones_v, cnt_acc.at[dst_idx.at[j]], add=True)

    def _chunk2(i, _):
        j0 = 2 * i
        pltpu.async_copy(table.at[src_idx.at[j0 + 1]], rows1, sem1)
        _consume(j0, rows0, sem0)

        @pl.when(j0 + 2 < nch)
        def _prefetch():
            pltpu.async_copy(table.at[src_idx.at[j0 + 2]], rows0, sem0)
        _consume(j0 + 1, rows1, sem1)
        return _

    pltpu.async_copy(table.at[src_idx.at[0]], rows0, sem0)
    lax.fori_loop(0, nch // 2, _chunk2, None)
    plsc.subcore_barrier()

    pltpu.sync_copy(acc.at[pl.ds(row0, TROW), :],
                    out.at[c, pl.ds(row0, TROW), :])
    if with_count:
        pltpu.sync_copy(cnt_acc.at[pl.ds(row0, TROW), :],
                        cnt_out.at[c, pl.ds(row0, TROW), :])

    @pl.when(s == 0)
    def _write_tail():
        pltpu.sync_copy(acc.at[pl.ds(TAIL0, TAIL), :],
                        out.at[c, pl.ds(TAIL0, TAIL), :])
        if with_count:
            pltpu.sync_copy(cnt_acc.at[pl.ds(TAIL0, TAIL), :],
                            cnt_out.at[c, pl.ds(TAIL0, TAIL), :])


@functools.lru_cache(maxsize=None)
def _make_sc_aggregate(o, with_count, k):
    nch = EPW // k
    mesh = plsc.VectorSubcoreMesh(
        core_axis_name="c", subcore_axis_name="s",
        num_cores=NC, num_subcores=NS)
    out_type = [jax.ShapeDtypeStruct((NC, N, o), jnp.float32)]
    scratch = [
        pltpu.VMEM((nch, k), jnp.int32),
        pltpu.VMEM((nch, k), jnp.int32),
        pltpu.VMEM((k, o), jnp.float32),
        pltpu.VMEM((k, o), jnp.float32),
    ]
    if with_count:
        out_type.append(jax.ShapeDtypeStruct((NC, N, CW), jnp.float32))
        scratch.append(pltpu.VMEM((k, CW), jnp.float32))
    scratch.append(pltpu.VMEM_SHARED((ACCR, o), jnp.float32))
    if with_count:
        scratch.append(pltpu.VMEM_SHARED((ACCR, CW), jnp.float32))
    scratch.extend([pltpu.SemaphoreType.DMA] * 2)
    return pl.kernel(
        functools.partial(_sc_body, with_count, o, k, nch),
        out_type=out_type,
        mesh=mesh,
        scratch_types=scratch,
        compiler_params=pltpu.CompilerParams(use_tc_tiling_on_sc=False),
        name=f"sc_seg_sum_{o}" + ("_cnt" if with_count else ""),
    )


_BR = 2000


def _dotT(a, w):
    return lax.dot_general(a, w, (((1,), (1,)), ((), ())),
                           preferred_element_type=jnp.float32)


def _tc_proj_body(x_ref, wl_ref, wr_ref, t_ref, r_ref):
    xb = x_ref[...]
    t_ref[...] = _dotT(xb, wl_ref[...])
    r_ref[...] = _dotT(xb, wr_ref[...])


def _combine(p_ref, cnt_ref, r_ref, b_ref):
    ssum = p_ref[0] + p_ref[1]
    cnts = cnt_ref[0, :, 0:1] + cnt_ref[1, :, 0:1]
    inv = 1.0 / jnp.maximum(cnts, 1.0)
    return jnp.maximum(ssum * inv + r_ref[...] + b_ref[...][None, :], 0.0)


def _tc_mid_body(p_ref, cnt_ref, r_ref, b_ref, wl_ref, wr_ref, t_ref, rr_ref):
    h = _combine(p_ref, cnt_ref, r_ref, b_ref)
    t_ref[...] = _dotT(h, wl_ref[...])
    rr_ref[...] = _dotT(h, wr_ref[...])


def _tc_dec_body(p_ref, cnt_ref, r_ref, b_ref,
                 d0w_ref, d0b_ref, d1w_ref, d1b_ref, d2w_ref, d2b_ref,
                 out_ref):
    h = _combine(p_ref, cnt_ref, r_ref, b_ref)
    h = jnp.maximum(_dotT(h, d0w_ref[...]) + d0b_ref[...][None, :], 0.0)
    h = jnp.maximum(_dotT(h, d1w_ref[...]) + d1b_ref[...][None, :], 0.0)
    out_ref[...] = jnp.maximum(
        _dotT(h, d2w_ref[...]) + d2b_ref[...][None, :], 0.0)


def _rows(shape):
    return pl.BlockSpec((_BR,) + shape[1:],
                        lambda i: (i,) + (0,) * (len(shape) - 1))


def _full(shape):
    return pl.BlockSpec(shape, lambda i: (0,) * len(shape))


def _part(o):
    return pl.BlockSpec((NC, _BR, o), lambda i: (0, i, 0))


def _tc_proj(x, wl, wr, o):
    return pl.pallas_call(
        _tc_proj_body,
        grid=(N // _BR,),
        in_specs=[_rows(x.shape), _full(wl.shape), _full(wr.shape)],
        out_specs=[_rows((N, o)), _rows((N, o))],
        out_shape=[jax.ShapeDtypeStruct((N, o), jnp.float32)] * 2,
    )(x, wl, wr)


def _tc_mid(p, cnt, r, b, wl, wr, o_in, o_out):
    return pl.pallas_call(
        _tc_mid_body,
        grid=(N // _BR,),
        in_specs=[_part(o_in), _part(CW), _rows(r.shape), _full(b.shape),
                  _full(wl.shape), _full(wr.shape)],
        out_specs=[_rows((N, o_out)), _rows((N, o_out))],
        out_shape=[jax.ShapeDtypeStruct((N, o_out), jnp.float32)] * 2,
    )(p, cnt, r, b, wl, wr)


def _tc_dec(p, cnt, r, b, d0w, d0b, d1w, d1b, d2w, d2b):
    return pl.pallas_call(
        _tc_dec_body,
        grid=(N // _BR,),
        in_specs=[_part(32), _part(CW), _rows(r.shape), _full(b.shape),
                  _full(d0w.shape), _full(d0b.shape),
                  _full(d1w.shape), _full(d1b.shape),
                  _full(d2w.shape), _full(d2b.shape)],
        out_specs=_rows((N, D)),
        out_shape=jax.ShapeDtypeStruct((N, D), jnp.float32),
    )(p, cnt, r, b, d0w, d0b, d1w, d1b, d2w, d2b)


def kernel(x, edge_index, e0Wl, e0Wr, e0b, e1Wl, e1Wr, e1b, e2Wl, e2Wr, e2b,
           d0W, d0b, d1W, d1b, d2W, d2b):
    pad = jnp.broadcast_to(jnp.array([[0], [N]], jnp.int32), (2, EP - E))
    epad = jnp.concatenate([edge_index, pad], axis=1)
    ei80 = epad.reshape(2, NW, EPW // 80, 80)
    ei128 = epad.reshape(2, NW, EPW // 128, 128)
    z128 = jnp.zeros((TROW + TAIL, 128), jnp.float32)
    zc = jnp.zeros((TROW + TAIL, CW), jnp.float32)

    t0, r0 = _tc_proj(x, e0Wl, e0Wr, 128)
    p0, cnt = _make_sc_aggregate(128, True, 80)(t0, ei80, z128, zc)
    t1, r1 = _tc_mid(p0, cnt, r0, e0b, e1Wl, e1Wr, 128, 64)
    (p1,) = _make_sc_aggregate(64, False, 128)(t1, ei128, z128[:, :64])
    t2, r2 = _tc_mid(p1, cnt, r1, e1b, e2Wl, e2Wr, 64, 32)
    (p2,) = _make_sc_aggregate(32, False, 128)(t2, ei128, z128[:, :32])
    return _tc_dec(p2, cnt, r2, e2b, d0W, d0b, d1W, d1b, d2W, d2b)


# speedup vs baseline: 1.1115x; 1.1115x over previous
import functools

import jax
import jax.numpy as jnp
from jax import lax
from jax.experimental import pallas as pl
from jax.experimental.pallas import tpu as pltpu
from jax.experimental.pallas import tpu_sc as plsc

N = 10000
E = 320000
D = 128

NC = 2
NS = 16
NW = NC * NS
EP = 327680
EPW = EP // NW
JUNK = 48
ACCR = N + JUNK
TROW = 624
TAIL = N - NS * TROW
TAIL0 = NS * TROW
CW = 8


def _sc_body(with_count, o, k, nch, *refs):
    if with_count:
        (table, ei, zeros_o, zeros_c, out, cnt_out,
         src_idx, dst_idx, rows0, rows1, ones_v, acc, cnt_acc,
         sem0, sem1) = refs
    else:
        (table, ei, zeros_o, out,
         src_idx, dst_idx, rows0, rows1, acc, sem0, sem1) = refs
    c = lax.axis_index("c")
    s = lax.axis_index("s")
    wid = c * NS + s

    pltpu.sync_copy(ei.at[0, wid], src_idx)
    pltpu.sync_copy(ei.at[1, wid], dst_idx)

    row0 = s * TROW
    pltpu.sync_copy(zeros_o.at[pl.ds(0, TROW), :],
                    acc.at[pl.ds(row0, TROW), :])
    if with_count:
        pltpu.sync_copy(zeros_c.at[pl.ds(0, TROW), :],
                        cnt_acc.at[pl.ds(row0, TROW), :])

        def _onerow(i, _):
            ones_v[i, :] = jnp.ones((CW,), jnp.float32)
            return _
        lax.fori_loop(0, k, _onerow, None)

    @pl.when(s == 0)
    def _zero_tail():
        pltpu.sync_copy(zeros_o.at[pl.ds(TROW, TAIL), :],
                        acc.at[pl.ds(TAIL0, TAIL), :])
        if with_count:
            pltpu.sync_copy(zeros_c.at[pl.ds(TROW, TAIL), :],
                            cnt_acc.at[pl.ds(TAIL0, TAIL), :])
    plsc.subcore_barrier()

    def _consume(j, buf, sem):
        pltpu.make_async_copy(table.at[src_idx.at[j]], buf, sem).wait()
        pltpu.sync_copy(buf, acc.at[dst_idx.at[j]], add=True)
        if with_count:
            pltpu.sync_copy(ones_v, cnt_acc.at[dst_idx.at[j]], add=True)

    def _chunk2(i, _):
        j0 = 2 * i
        pltpu.async_copy(table.at[src_idx.at[j0 + 1]], rows1, sem1)
        _consume(j0, rows0, sem0)

        @pl.when(j0 + 2 < nch)
        def _prefetch():
            pltpu.async_copy(table.at[src_idx.at[j0 + 2]], rows0, sem0)
        _consume(j0 + 1, rows1, sem1)
        return _

    pltpu.async_copy(table.at[src_idx.at[0]], rows0, sem0)
    lax.fori_loop(0, nch // 2, _chunk2, None)
    plsc.subcore_barrier()

    pltpu.sync_copy(acc.at[pl.ds(row0, TROW), :],
                    out.at[c, pl.ds(row0, TROW), :])
    if with_count:
        pltpu.sync_copy(cnt_acc.at[pl.ds(row0, TROW), :],
                        cnt_out.at[c, pl.ds(row0, TROW), :])

    @pl.when(s == 0)
    def _write_tail():
        pltpu.sync_copy(acc.at[pl.ds(TAIL0, TAIL), :],
                        out.at[c, pl.ds(TAIL0, TAIL), :])
        if with_count:
            pltpu.sync_copy(cnt_acc.at[pl.ds(TAIL0, TAIL), :],
                            cnt_out.at[c, pl.ds(TAIL0, TAIL), :])


@functools.lru_cache(maxsize=None)
def _make_sc_aggregate(o, with_count, k):
    nch = EPW // k
    mesh = plsc.VectorSubcoreMesh(
        core_axis_name="c", subcore_axis_name="s",
        num_cores=NC, num_subcores=NS)
    out_type = [jax.ShapeDtypeStruct((NC, N, o), jnp.float32)]
    scratch = [
        pltpu.VMEM((nch, k), jnp.int32),
        pltpu.VMEM((nch, k), jnp.int32),
        pltpu.VMEM((k, o), jnp.float32),
        pltpu.VMEM((k, o), jnp.float32),
    ]
    if with_count:
        out_type.append(jax.ShapeDtypeStruct((NC, N, CW), jnp.float32))
        scratch.append(pltpu.VMEM((k, CW), jnp.float32))
    scratch.append(pltpu.VMEM_SHARED((ACCR, o), jnp.float32))
    if with_count:
        scratch.append(pltpu.VMEM_SHARED((ACCR, CW), jnp.float32))
    scratch.extend([pltpu.SemaphoreType.DMA] * 2)
    return pl.kernel(
        functools.partial(_sc_body, with_count, o, k, nch),
        out_type=out_type,
        mesh=mesh,
        scratch_types=scratch,
        compiler_params=pltpu.CompilerParams(use_tc_tiling_on_sc=False),
        name=f"sc_seg_sum_{o}" + ("_cnt" if with_count else ""),
    )


_BR = 2000


def _dotT(a, w):
    return lax.dot_general(a, w, (((1,), (1,)), ((), ())),
                           preferred_element_type=jnp.float32)


def _tc_proj_body(x_ref, wl_ref, wr_ref, t_ref, r_ref):
    xb = x_ref[...]
    t_ref[...] = _dotT(xb, wl_ref[...])
    r_ref[...] = _dotT(xb, wr_ref[...])


def _combine(p_ref, cnt_ref, r_ref, b_ref):
    ssum = p_ref[0] + p_ref[1]
    cnts = cnt_ref[0, :, 0:1] + cnt_ref[1, :, 0:1]
    inv = 1.0 / jnp.maximum(cnts, 1.0)
    return jnp.maximum(ssum * inv + r_ref[...] + b_ref[...][None, :], 0.0)


def _tc_mid_body(p_ref, cnt_ref, r_ref, b_ref, wl_ref, wr_ref, t_ref, rr_ref):
    h = _combine(p_ref, cnt_ref, r_ref, b_ref)
    t_ref[...] = _dotT(h, wl_ref[...])
    rr_ref[...] = _dotT(h, wr_ref[...])


def _tc_dec_body(p_ref, cnt_ref, r_ref, b_ref,
                 d0w_ref, d0b_ref, d1w_ref, d1b_ref, d2w_ref, d2b_ref,
                 out_ref):
    h = _combine(p_ref, cnt_ref, r_ref, b_ref)
    h = jnp.maximum(_dotT(h, d0w_ref[...]) + d0b_ref[...][None, :], 0.0)
    h = jnp.maximum(_dotT(h, d1w_ref[...]) + d1b_ref[...][None, :], 0.0)
    out_ref[...] = jnp.maximum(
        _dotT(h, d2w_ref[...]) + d2b_ref[...][None, :], 0.0)


def _rows(shape):
    return pl.BlockSpec((_BR,) + shape[1:],
                        lambda i: (i,) + (0,) * (len(shape) - 1))


def _full(shape):
    return pl.BlockSpec(shape, lambda i: (0,) * len(shape))


def _part(o):
    return pl.BlockSpec((NC, _BR, o), lambda i: (0, i, 0))


def _tc_proj(x, wl, wr, o):
    return pl.pallas_call(
        _tc_proj_body,
        grid=(N // _BR,),
        in_specs=[_rows(x.shape), _full(wl.shape), _full(wr.shape)],
        out_specs=[_rows((N, o)), _rows((N, o))],
        out_shape=[jax.ShapeDtypeStruct((N, o), jnp.float32)] * 2,
    )(x, wl, wr)


def _tc_mid(p, cnt, r, b, wl, wr, o_in, o_out):
    return pl.pallas_call(
        _tc_mid_body,
        grid=(N // _BR,),
        in_specs=[_part(o_in), _part(CW), _rows(r.shape), _full(b.shape),
                  _full(wl.shape), _full(wr.shape)],
        out_specs=[_rows((N, o_out)), _rows((N, o_out))],
        out_shape=[jax.ShapeDtypeStruct((N, o_out), jnp.float32)] * 2,
    )(p, cnt, r, b, wl, wr)


def _tc_dec(p, cnt, r, b, d0w, d0b, d1w, d1b, d2w, d2b):
    return pl.pallas_call(
        _tc_dec_body,
        grid=(N // _BR,),
        in_specs=[_part(32), _part(CW), _rows(r.shape), _full(b.shape),
                  _full(d0w.shape), _full(d0b.shape),
                  _full(d1w.shape), _full(d1b.shape),
                  _full(d2w.shape), _full(d2b.shape)],
        out_specs=_rows((N, D)),
        out_shape=jax.ShapeDtypeStruct((N, D), jnp.float32),
    )(p, cnt, r, b, d0w, d0b, d1w, d1b, d2w, d2b)


def kernel(x, edge_index, e0Wl, e0Wr, e0b, e1Wl, e1Wr, e1b, e2Wl, e2Wr, e2b,
           d0W, d0b, d1W, d1b, d2W, d2b):
    npad = EPW - E // NW
    pad_dst = N + (jnp.arange(npad, dtype=jnp.int32) % JUNK)
    pad = jnp.stack([jnp.zeros((npad,), jnp.int32), pad_dst])
    pad = jnp.broadcast_to(pad[:, None, :], (2, NW, npad))
    epad = jnp.concatenate(
        [edge_index.reshape(2, NW, E // NW), pad], axis=2)
    ei80 = epad.reshape(2, NW, EPW // 80, 80)
    ei128 = epad.reshape(2, NW, EPW // 128, 128)
    z128 = jnp.zeros((TROW + TAIL, 128), jnp.float32)
    zc = jnp.zeros((TROW + TAIL, CW), jnp.float32)

    t0, r0 = _tc_proj(x, e0Wl, e0Wr, 128)
    p0, cnt = _make_sc_aggregate(128, True, 80)(t0, ei80, z128, zc)
    t1, r1 = _tc_mid(p0, cnt, r0, e0b, e1Wl, e1Wr, 128, 64)
    (p1,) = _make_sc_aggregate(64, False, 128)(t1, ei128, z128[:, :64])
    t2, r2 = _tc_mid(p1, cnt, r1, e1b, e2Wl, e2Wr, 64, 32)
    (p2,) = _make_sc_aggregate(32, False, 128)(t2, ei128, z128[:, :32])
    return _tc_dec(p2, cnt, r2, e2b, d0W, d0b, d1W, d1b, d2W, d2b)


# speedup vs baseline: 2.4964x; 2.2460x over previous
import functools

import jax
import jax.numpy as jnp
from jax import lax
from jax.experimental import pallas as pl
from jax.experimental.pallas import tpu as pltpu
from jax.experimental.pallas import tpu_sc as plsc

N = 10000
E = 320000
D = 128

NC = 2
NS = 16
NW = NC * NS
EPW = E // NW
ACCR = N
TROW = 624
TAIL = N - NS * TROW
TAIL0 = NS * TROW
CW = 8


def _sc_body(with_count, o, k, nch, *refs):
    if with_count:
        (table, ei, zeros_o, zeros_c, out, cnt_out,
         src_idx, dst_idx, rows0, rows1, ones_v, acc, cnt_acc,
         sem0, sem1) = refs
    else:
        (table, ei, zeros_o, out,
         src_idx, dst_idx, rows0, rows1, acc, sem0, sem1) = refs
    c = lax.axis_index("c")
    s = lax.axis_index("s")
    wid = c * NS + s

    pltpu.sync_copy(ei.at[0, wid], src_idx)
    pltpu.sync_copy(ei.at[1, wid], dst_idx)

    row0 = s * TROW
    pltpu.sync_copy(zeros_o.at[pl.ds(0, TROW), :],
                    acc.at[pl.ds(row0, TROW), :])
    if with_count:
        pltpu.sync_copy(zeros_c.at[pl.ds(0, TROW), :],
                        cnt_acc.at[pl.ds(row0, TROW), :])

        def _onerow(i, _):
            ones_v[i, :] = jnp.ones((CW,), jnp.float32)
            return _
        lax.fori_loop(0, k, _onerow, None)

    @pl.when(s == 0)
    def _zero_tail():
        pltpu.sync_copy(zeros_o.at[pl.ds(TROW, TAIL), :],
                        acc.at[pl.ds(TAIL0, TAIL), :])
        if with_count:
            pltpu.sync_copy(zeros_c.at[pl.ds(TROW, TAIL), :],
                            cnt_acc.at[pl.ds(TAIL0, TAIL), :])
    plsc.subcore_barrier()

    def _consume(j, buf, sem):
        pltpu.make_async_copy(table.at[src_idx.at[j]], buf, sem).wait()
        pltpu.sync_copy(buf, acc.at[dst_idx.at[j]], add=True)
        if with_count:
            pltpu.sync_copy(ones_v, cnt_acc.at[dst_idx.at[j]], add=True)

    def _chunk2(i, _):
        j0 = 2 * i
        pltpu.async_copy(table.at[src_idx.at[j0 + 1]], rows1, sem1)
        _consume(j0, rows0, sem0)
        pltpu.async_copy(table.at[src_idx.at[j0 + 2]], rows0, sem0)
        _consume(j0 + 1, rows1, sem1)
        return _

    pltpu.async_copy(table.at[src_idx.at[0]], rows0, sem0)
    lax.fori_loop(0, (nch - 1) // 2, _chunk2, None)
    _consume(nch - 1, rows0, sem0)
    plsc.subcore_barrier()

    pltpu.sync_copy(acc.at[pl.ds(row0, TROW), :],
                    out.at[c, pl.ds(row0, TROW), :])
    if with_count:
        pltpu.sync_copy(cnt_acc.at[pl.ds(row0, TROW), :],
                        cnt_out.at[c, pl.ds(row0, TROW), :])

    @pl.when(s == 0)
    def _write_tail():
        pltpu.sync_copy(acc.at[pl.ds(TAIL0, TAIL), :],
                        out.at[c, pl.ds(TAIL0, TAIL), :])
        if with_count:
            pltpu.sync_copy(cnt_acc.at[pl.ds(TAIL0, TAIL), :],
                            cnt_out.at[c, pl.ds(TAIL0, TAIL), :])


@functools.lru_cache(maxsize=None)
def _make_sc_aggregate(o, with_count, k):
    nch = EPW // k
    mesh = plsc.VectorSubcoreMesh(
        core_axis_name="c", subcore_axis_name="s",
        num_cores=NC, num_subcores=NS)
    out_type = [jax.ShapeDtypeStruct((NC, N, o), jnp.float32)]
    scratch = [
        pltpu.VMEM((nch, k), jnp.int32),
        pltpu.VMEM((nch, k), jnp.int32),
        pltpu.VMEM((k, o), jnp.float32),
        pltpu.VMEM((k, o), jnp.float32),
    ]
    if with_count:
        out_type.append(jax.ShapeDtypeStruct((NC, N, CW), jnp.float32))
        scratch.append(pltpu.VMEM((k, CW), jnp.float32))
    scratch.append(pltpu.VMEM_SHARED((ACCR, o), jnp.float32))
    if with_count:
        scratch.append(pltpu.VMEM_SHARED((ACCR, CW), jnp.float32))
    scratch.extend([pltpu.SemaphoreType.DMA] * 2)
    return pl.kernel(
        functools.partial(_sc_body, with_count, o, k, nch),
        out_type=out_type,
        mesh=mesh,
        scratch_types=scratch,
        compiler_params=pltpu.CompilerParams(use_tc_tiling_on_sc=False),
        name=f"sc_seg_sum_{o}" + ("_cnt" if with_count else ""),
    )


_BR = 2000


def _dotT(a, w):
    return lax.dot_general(a, w, (((1,), (1,)), ((), ())),
                           preferred_element_type=jnp.float32)


def _tc_proj_body(x_ref, wl_ref, wr_ref, t_ref, r_ref):
    xb = x_ref[...]
    t_ref[...] = _dotT(xb, wl_ref[...])
    r_ref[...] = _dotT(xb, wr_ref[...])


def _combine(p_ref, cnt_ref, r_ref, b_ref):
    ssum = p_ref[0] + p_ref[1]
    cnts = cnt_ref[0, :, 0:1] + cnt_ref[1, :, 0:1]
    inv = 1.0 / jnp.maximum(cnts, 1.0)
    return jnp.maximum(ssum * inv + r_ref[...] + b_ref[...][None, :], 0.0)


def _tc_mid_body(p_ref, cnt_ref, r_ref, b_ref, wl_ref, wr_ref, t_ref, rr_ref):
    h = _combine(p_ref, cnt_ref, r_ref, b_ref)
    t_ref[...] = _dotT(h, wl_ref[...])
    rr_ref[...] = _dotT(h, wr_ref[...])


def _tc_dec_body(p_ref, cnt_ref, r_ref, b_ref,
                 d0w_ref, d0b_ref, d1w_ref, d1b_ref, d2w_ref, d2b_ref,
                 out_ref):
    h = _combine(p_ref, cnt_ref, r_ref, b_ref)
    h = jnp.maximum(_dotT(h, d0w_ref[...]) + d0b_ref[...][None, :], 0.0)
    h = jnp.maximum(_dotT(h, d1w_ref[...]) + d1b_ref[...][None, :], 0.0)
    out_ref[...] = jnp.maximum(
        _dotT(h, d2w_ref[...]) + d2b_ref[...][None, :], 0.0)


def _rows(shape):
    return pl.BlockSpec((_BR,) + shape[1:],
                        lambda i: (i,) + (0,) * (len(shape) - 1))


def _full(shape):
    return pl.BlockSpec(shape, lambda i: (0,) * len(shape))


def _part(o):
    return pl.BlockSpec((NC, _BR, o), lambda i: (0, i, 0))


def _tc_proj(x, wl, wr, o):
    return pl.pallas_call(
        _tc_proj_body,
        grid=(N // _BR,),
        in_specs=[_rows(x.shape), _full(wl.shape), _full(wr.shape)],
        out_specs=[_rows((N, o)), _rows((N, o))],
        out_shape=[jax.ShapeDtypeStruct((N, o), jnp.float32)] * 2,
    )(x, wl, wr)


def _tc_mid(p, cnt, r, b, wl, wr, o_in, o_out):
    return pl.pallas_call(
        _tc_mid_body,
        grid=(N // _BR,),
        in_specs=[_part(o_in), _part(CW), _rows(r.shape), _full(b.shape),
                  _full(wl.shape), _full(wr.shape)],
        out_specs=[_rows((N, o_out)), _rows((N, o_out))],
        out_shape=[jax.ShapeDtypeStruct((N, o_out), jnp.float32)] * 2,
    )(p, cnt, r, b, wl, wr)


def _tc_dec(p, cnt, r, b, d0w, d0b, d1w, d1b, d2w, d2b):
    return pl.pallas_call(
        _tc_dec_body,
        grid=(N // _BR,),
        in_specs=[_part(32), _part(CW), _rows(r.shape), _full(b.shape),
                  _full(d0w.shape), _full(d0b.shape),
                  _full(d1w.shape), _full(d1b.shape),
                  _full(d2w.shape), _full(d2b.shape)],
        out_specs=_rows((N, D)),
        out_shape=jax.ShapeDtypeStruct((N, D), jnp.float32),
    )(p, cnt, r, b, d0w, d0b, d1w, d1b, d2w, d2b)


def kernel(x, edge_index, e0Wl, e0Wr, e0b, e1Wl, e1Wr, e1b, e2Wl, e2Wr, e2b,
           d0W, d0b, d1W, d1b, d2W, d2b):
    ei80 = edge_index.reshape(2, NW, EPW // 80, 80)
    z128 = jnp.zeros((TROW + TAIL, 128), jnp.float32)
    zc = jnp.zeros((TROW + TAIL, CW), jnp.float32)

    t0, r0 = _tc_proj(x, e0Wl, e0Wr, 128)
    p0, cnt = _make_sc_aggregate(128, True, 80)(t0, ei80, z128, zc)
    t1, r1 = _tc_mid(p0, cnt, r0, e0b, e1Wl, e1Wr, 128, 64)
    (p1,) = _make_sc_aggregate(64, False, 80)(t1, ei80, z128[:, :64])
    t2, r2 = _tc_mid(p1, cnt, r1, e1b, e2Wl, e2Wr, 64, 32)
    (p2,) = _make_sc_aggregate(32, False, 80)(t2, ei80, z128[:, :32])
    return _tc_dec(p2, cnt, r2, e2b, d0W, d0b, d1W, d1b, d2W, d2b)


# speedup vs baseline: 2.6079x; 1.0447x over previous
import functools

import jax
import jax.numpy as jnp
from jax import lax
from jax.experimental import pallas as pl
from jax.experimental.pallas import tpu as pltpu
from jax.experimental.pallas import tpu_sc as plsc

N = 10000
E = 320000
D = 128

NC = 2
NS = 16
NW = NC * NS
EPW = E // NW
ACCR = N
TROW = 624
TAIL = N - NS * TROW
TAIL0 = NS * TROW
CW = 8


def _sc_body(with_count, o, k, nch, *refs):
    if with_count:
        (table, ei, zeros_o, zeros_c, out, cnt_out,
         src_idx, dst_idx, rows0, rows1, ones_v, acc, cnt_acc,
         sem0, sem1, semc) = refs
    else:
        (table, ei, zeros_o, out,
         src_idx, dst_idx, rows0, rows1, acc, sem0, sem1) = refs
    c = lax.axis_index("c")
    s = lax.axis_index("s")
    wid = c * NS + s

    pltpu.sync_copy(ei.at[0, wid], src_idx)
    pltpu.sync_copy(ei.at[1, wid], dst_idx)

    row0 = s * TROW
    pltpu.sync_copy(zeros_o.at[pl.ds(0, TROW), :],
                    acc.at[pl.ds(row0, TROW), :])
    if with_count:
        pltpu.sync_copy(zeros_c.at[pl.ds(0, TROW), :],
                        cnt_acc.at[pl.ds(row0, TROW), :])

        def _onerow(i, _):
            ones_v[i, :] = jnp.ones((CW,), jnp.float32)
            return _
        lax.fori_loop(0, k, _onerow, None)

    @pl.when(s == 0)
    def _zero_tail():
        pltpu.sync_copy(zeros_o.at[pl.ds(TROW, TAIL), :],
                        acc.at[pl.ds(TAIL0, TAIL), :])
        if with_count:
            pltpu.sync_copy(zeros_c.at[pl.ds(TROW, TAIL), :],
                            cnt_acc.at[pl.ds(TAIL0, TAIL), :])
    plsc.subcore_barrier()

    def _consume(j, buf, sem):
        pltpu.make_async_copy(table.at[src_idx.at[j]], buf, sem).wait()
        pltpu.sync_copy(buf, acc.at[dst_idx.at[j]], add=True)
        if with_count:
            pltpu.async_copy(ones_v, cnt_acc.at[dst_idx.at[j]], semc,
                             add=True)

    def _chunk2(i, _):
        j0 = 2 * i
        pltpu.async_copy(table.at[src_idx.at[j0 + 1]], rows1, sem1)
        _consume(j0, rows0, sem0)
        pltpu.async_copy(table.at[src_idx.at[j0 + 2]], rows0, sem0)
        _consume(j0 + 1, rows1, sem1)
        return _

    pltpu.async_copy(table.at[src_idx.at[0]], rows0, sem0)
    lax.fori_loop(0, (nch - 1) // 2, _chunk2, None)
    if nch % 2:
        _consume(nch - 1, rows0, sem0)
    else:
        pltpu.async_copy(table.at[src_idx.at[nch - 1]], rows1, sem1)
        _consume(nch - 2, rows0, sem0)
        _consume(nch - 1, rows1, sem1)
    if with_count:
        def _drain(i, _):
            pltpu.make_async_copy(ones_v, cnt_acc.at[dst_idx.at[0]],
                                  semc).wait()
            return _
        lax.fori_loop(0, nch, _drain, None)
    plsc.subcore_barrier()

    pltpu.sync_copy(acc.at[pl.ds(row0, TROW), :],
                    out.at[c, pl.ds(row0, TROW), :])
    if with_count:
        pltpu.sync_copy(cnt_acc.at[pl.ds(row0, TROW), :],
                        cnt_out.at[c, pl.ds(row0, TROW), :])

    @pl.when(s == 0)
    def _write_tail():
        pltpu.sync_copy(acc.at[pl.ds(TAIL0, TAIL), :],
                        out.at[c, pl.ds(TAIL0, TAIL), :])
        if with_count:
            pltpu.sync_copy(cnt_acc.at[pl.ds(TAIL0, TAIL), :],
                            cnt_out.at[c, pl.ds(TAIL0, TAIL), :])


@functools.lru_cache(maxsize=None)
def _make_sc_aggregate(o, with_count, k):
    nch = EPW // k
    mesh = plsc.VectorSubcoreMesh(
        core_axis_name="c", subcore_axis_name="s",
        num_cores=NC, num_subcores=NS)
    out_type = [jax.ShapeDtypeStruct((NC, N, o), jnp.float32)]
    scratch = [
        pltpu.VMEM((nch, k), jnp.int32),
        pltpu.VMEM((nch, k), jnp.int32),
        pltpu.VMEM((k, o), jnp.float32),
        pltpu.VMEM((k, o), jnp.float32),
    ]
    if with_count:
        out_type.append(jax.ShapeDtypeStruct((NC, N, CW), jnp.float32))
        scratch.append(pltpu.VMEM((k, CW), jnp.float32))
    scratch.append(pltpu.VMEM_SHARED((ACCR, o), jnp.float32))
    if with_count:
        scratch.append(pltpu.VMEM_SHARED((ACCR, CW), jnp.float32))
    scratch.extend([pltpu.SemaphoreType.DMA] * (3 if with_count else 2))
    return pl.kernel(
        functools.partial(_sc_body, with_count, o, k, nch),
        out_type=out_type,
        mesh=mesh,
        scratch_types=scratch,
        compiler_params=pltpu.CompilerParams(use_tc_tiling_on_sc=False),
        name=f"sc_seg_sum_{o}" + ("_cnt" if with_count else ""),
    )


_BR = 2000


def _dotT(a, w):
    return lax.dot_general(a, w, (((1,), (1,)), ((), ())),
                           preferred_element_type=jnp.float32)


def _tc_proj_body(x_ref, wl_ref, wr_ref, t_ref, r_ref):
    xb = x_ref[...]
    t_ref[...] = _dotT(xb, wl_ref[...])
    r_ref[...] = _dotT(xb, wr_ref[...])


def _combine(p_ref, cnt_ref, r_ref, b_ref):
    ssum = p_ref[0] + p_ref[1]
    cnts = cnt_ref[0, :, 0:1] + cnt_ref[1, :, 0:1]
    inv = 1.0 / jnp.maximum(cnts, 1.0)
    return jnp.maximum(ssum * inv + r_ref[...] + b_ref[...][None, :], 0.0)


def _tc_mid_body(p_ref, cnt_ref, r_ref, b_ref, wl_ref, wr_ref, t_ref, rr_ref):
    h = _combine(p_ref, cnt_ref, r_ref, b_ref)
    t_ref[...] = _dotT(h, wl_ref[...])
    rr_ref[...] = _dotT(h, wr_ref[...])


def _tc_dec_body(p_ref, cnt_ref, r_ref, b_ref,
                 d0w_ref, d0b_ref, d1w_ref, d1b_ref, d2w_ref, d2b_ref,
                 out_ref):
    h = _combine(p_ref, cnt_ref, r_ref, b_ref)
    h = jnp.maximum(_dotT(h, d0w_ref[...]) + d0b_ref[...][None, :], 0.0)
    h = jnp.maximum(_dotT(h, d1w_ref[...]) + d1b_ref[...][None, :], 0.0)
    out_ref[...] = jnp.maximum(
        _dotT(h, d2w_ref[...]) + d2b_ref[...][None, :], 0.0)


def _rows(shape):
    return pl.BlockSpec((_BR,) + shape[1:],
                        lambda i: (i,) + (0,) * (len(shape) - 1))


def _full(shape):
    return pl.BlockSpec(shape, lambda i: (0,) * len(shape))


def _part(o):
    return pl.BlockSpec((NC, _BR, o), lambda i: (0, i, 0))


def _tc_proj(x, wl, wr, o):
    return pl.pallas_call(
        _tc_proj_body,
        grid=(N // _BR,),
        in_specs=[_rows(x.shape), _full(wl.shape), _full(wr.shape)],
        out_specs=[_rows((N, o)), _rows((N, o))],
        out_shape=[jax.ShapeDtypeStruct((N, o), jnp.float32)] * 2,
    )(x, wl, wr)


def _tc_mid(p, cnt, r, b, wl, wr, o_in, o_out):
    return pl.pallas_call(
        _tc_mid_body,
        grid=(N // _BR,),
        in_specs=[_part(o_in), _part(CW), _rows(r.shape), _full(b.shape),
                  _full(wl.shape), _full(wr.shape)],
        out_specs=[_rows((N, o_out)), _rows((N, o_out))],
        out_shape=[jax.ShapeDtypeStruct((N, o_out), jnp.float32)] * 2,
    )(p, cnt, r, b, wl, wr)


def _tc_dec(p, cnt, r, b, d0w, d0b, d1w, d1b, d2w, d2b):
    return pl.pallas_call(
        _tc_dec_body,
        grid=(N // _BR,),
        in_specs=[_part(32), _part(CW), _rows(r.shape), _full(b.shape),
                  _full(d0w.shape), _full(d0b.shape),
                  _full(d1w.shape), _full(d1b.shape),
                  _full(d2w.shape), _full(d2b.shape)],
        out_specs=_rows((N, D)),
        out_shape=jax.ShapeDtypeStruct((N, D), jnp.float32),
    )(p, cnt, r, b, d0w, d0b, d1w, d1b, d2w, d2b)


def kernel(x, edge_index, e0Wl, e0Wr, e0b, e1Wl, e1Wr, e1b, e2Wl, e2Wr, e2b,
           d0W, d0b, d1W, d1b, d2W, d2b):
    ei80 = edge_index.reshape(2, NW, EPW // 80, 80)
    ei100 = edge_index.reshape(2, NW, EPW // 100, 100)
    z128 = jnp.zeros((TROW + TAIL, 128), jnp.float32)
    zc = jnp.zeros((TROW + TAIL, CW), jnp.float32)

    t0, r0 = _tc_proj(x, e0Wl, e0Wr, 128)
    p0, cnt = _make_sc_aggregate(128, True, 80)(t0, ei80, z128, zc)
    t1, r1 = _tc_mid(p0, cnt, r0, e0b, e1Wl, e1Wr, 128, 64)
    (p1,) = _make_sc_aggregate(64, False, 100)(t1, ei100, z128[:, :64])
    t2, r2 = _tc_mid(p1, cnt, r1, e1b, e2Wl, e2Wr, 64, 32)
    (p2,) = _make_sc_aggregate(32, False, 100)(t2, ei100, z128[:, :32])
    return _tc_dec(p2, cnt, r2, e2b, d0W, d0b, d1W, d1b, d2W, d2b)


# speedup vs baseline: 2.8495x; 1.0927x over previous
import functools

import jax
import jax.numpy as jnp
from jax import lax
from jax.experimental import pallas as pl
from jax.experimental.pallas import tpu as pltpu
from jax.experimental.pallas import tpu_sc as plsc

N = 10000
E = 320000
D = 128

NC = 2
NS = 16
NW = NC * NS
EPW = E // NW
ACCR = N
TROW = 624
TAIL = N - NS * TROW
TAIL0 = NS * TROW
CW = 8


def _sc_body(with_count, o, k, nch, nbuf, *refs):
    if with_count:
        (table, ei, zeros_o, zeros_c, out, cnt_out,
         src_idx, dst_idx, rows0, rows1, ones_v, acc, cnt_acc,
         sem0, sem1, semc) = refs
    else:
        (table, ei, zeros_o, out, src_idx, dst_idx,
         rows0, rows1, rows2, acc, sem0, sem1, sem2) = refs
    c = lax.axis_index("c")
    s = lax.axis_index("s")
    wid = c * NS + s

    pltpu.sync_copy(ei.at[0, wid], src_idx)
    pltpu.sync_copy(ei.at[1, wid], dst_idx)

    row0 = s * TROW
    pltpu.sync_copy(zeros_o.at[pl.ds(0, TROW), :],
                    acc.at[pl.ds(row0, TROW), :])
    if with_count:
        pltpu.sync_copy(zeros_c.at[pl.ds(0, TROW), :],
                        cnt_acc.at[pl.ds(row0, TROW), :])

        def _onerow(i, _):
            ones_v[i, :] = jnp.ones((CW,), jnp.float32)
            return _
        lax.fori_loop(0, k, _onerow, None)

    @pl.when(s == 0)
    def _zero_tail():
        pltpu.sync_copy(zeros_o.at[pl.ds(TROW, TAIL), :],
                        acc.at[pl.ds(TAIL0, TAIL), :])
        if with_count:
            pltpu.sync_copy(zeros_c.at[pl.ds(TROW, TAIL), :],
                            cnt_acc.at[pl.ds(TAIL0, TAIL), :])
    plsc.subcore_barrier()

    def _consume(j, buf, sem):
        pltpu.make_async_copy(table.at[src_idx.at[j]], buf, sem).wait()
        pltpu.sync_copy(buf, acc.at[dst_idx.at[j]], add=True)
        if with_count:
            pltpu.async_copy(ones_v, cnt_acc.at[dst_idx.at[j]], semc,
                             add=True)

    def _gather(j, buf, sem):
        pltpu.async_copy(table.at[src_idx.at[j]], buf, sem)

    if nbuf == 2:
        def _chunk2(i, _):
            j0 = 2 * i
            _gather(j0 + 1, rows1, sem1)
            _consume(j0, rows0, sem0)
            _gather(j0 + 2, rows0, sem0)
            _consume(j0 + 1, rows1, sem1)
            return _

        _gather(0, rows0, sem0)
        lax.fori_loop(0, (nch - 1) // 2, _chunk2, None)
        _consume(nch - 1, rows0, sem0)
    else:
        def _chunk3(i, _):
            j0 = 3 * i
            _gather(j0 + 2, rows2, sem2)
            _consume(j0, rows0, sem0)
            _gather(j0 + 3, rows0, sem0)
            _consume(j0 + 1, rows1, sem1)
            _gather(j0 + 4, rows1, sem1)
            _consume(j0 + 2, rows2, sem2)
            return _

        _gather(0, rows0, sem0)
        _gather(1, rows1, sem1)
        lax.fori_loop(0, (nch - 4) // 3, _chunk3, None)
        _gather(nch - 2, rows2, sem2)
        _consume(nch - 4, rows0, sem0)
        _gather(nch - 1, rows0, sem0)
        _consume(nch - 3, rows1, sem1)
        _consume(nch - 2, rows2, sem2)
        _consume(nch - 1, rows0, sem0)
    if with_count:
        def _drain(i, _):
            pltpu.make_async_copy(ones_v, cnt_acc.at[dst_idx.at[0]],
                                  semc).wait()
            return _
        lax.fori_loop(0, nch, _drain, None)
    plsc.subcore_barrier()

    pltpu.sync_copy(acc.at[pl.ds(row0, TROW), :],
                    out.at[c, pl.ds(row0, TROW), :])
    if with_count:
        pltpu.sync_copy(cnt_acc.at[pl.ds(row0, TROW), :],
                        cnt_out.at[c, pl.ds(row0, TROW), :])

    @pl.when(s == 0)
    def _write_tail():
        pltpu.sync_copy(acc.at[pl.ds(TAIL0, TAIL), :],
                        out.at[c, pl.ds(TAIL0, TAIL), :])
        if with_count:
            pltpu.sync_copy(cnt_acc.at[pl.ds(TAIL0, TAIL), :],
                            cnt_out.at[c, pl.ds(TAIL0, TAIL), :])


@functools.lru_cache(maxsize=None)
def _make_sc_aggregate(o, with_count, k):
    nch = EPW // k
    nbuf = 2 if with_count else 3
    mesh = plsc.VectorSubcoreMesh(
        core_axis_name="c", subcore_axis_name="s",
        num_cores=NC, num_subcores=NS)
    out_type = [jax.ShapeDtypeStruct((NC, N, o), jnp.float32)]
    scratch = [
        pltpu.VMEM((nch, k), jnp.int32),
        pltpu.VMEM((nch, k), jnp.int32),
    ]
    scratch += [pltpu.VMEM((k, o), jnp.float32)] * nbuf
    if with_count:
        out_type.append(jax.ShapeDtypeStruct((NC, N, CW), jnp.float32))
        scratch.append(pltpu.VMEM((k, CW), jnp.float32))
    scratch.append(pltpu.VMEM_SHARED((ACCR, o), jnp.float32))
    if with_count:
        scratch.append(pltpu.VMEM_SHARED((ACCR, CW), jnp.float32))
    scratch.extend([pltpu.SemaphoreType.DMA] * 3)
    return pl.kernel(
        functools.partial(_sc_body, with_count, o, k, nch, nbuf),
        out_type=out_type,
        mesh=mesh,
        scratch_types=scratch,
        compiler_params=pltpu.CompilerParams(use_tc_tiling_on_sc=False),
        name=f"sc_seg_sum_{o}" + ("_cnt" if with_count else ""),
    )


_BR = 2000


def _dotT(a, w):
    return lax.dot_general(a, w, (((1,), (1,)), ((), ())),
                           preferred_element_type=jnp.float32)


def _tc_proj_body(x_ref, wl_ref, wr_ref, t_ref, r_ref):
    xb = x_ref[...]
    t_ref[...] = _dotT(xb, wl_ref[...])
    r_ref[...] = _dotT(xb, wr_ref[...])


def _combine(p_ref, cnt_ref, r_ref, b_ref):
    ssum = p_ref[0] + p_ref[1]
    cnts = cnt_ref[0, :, 0:1] + cnt_ref[1, :, 0:1]
    inv = 1.0 / jnp.maximum(cnts, 1.0)
    return jnp.maximum(ssum * inv + r_ref[...] + b_ref[...][None, :], 0.0)


def _tc_mid_body(p_ref, cnt_ref, r_ref, b_ref, wl_ref, wr_ref, t_ref, rr_ref):
    h = _combine(p_ref, cnt_ref, r_ref, b_ref)
    t_ref[...] = _dotT(h, wl_ref[...])
    rr_ref[...] = _dotT(h, wr_ref[...])


def _tc_dec_body(p_ref, cnt_ref, r_ref, b_ref,
                 d0w_ref, d0b_ref, d1w_ref, d1b_ref, d2w_ref, d2b_ref,
                 out_ref):
    h = _combine(p_ref, cnt_ref, r_ref, b_ref)
    h = jnp.maximum(_dotT(h, d0w_ref[...]) + d0b_ref[...][None, :], 0.0)
    h = jnp.maximum(_dotT(h, d1w_ref[...]) + d1b_ref[...][None, :], 0.0)
    out_ref[...] = jnp.maximum(
        _dotT(h, d2w_ref[...]) + d2b_ref[...][None, :], 0.0)


def _rows(shape):
    return pl.BlockSpec((_BR,) + shape[1:],
                        lambda i: (i,) + (0,) * (len(shape) - 1))


def _full(shape):
    return pl.BlockSpec(shape, lambda i: (0,) * len(shape))


def _part(o):
    return pl.BlockSpec((NC, _BR, o), lambda i: (0, i, 0))


def _tc_proj(x, wl, wr, o):
    return pl.pallas_call(
        _tc_proj_body,
        grid=(N // _BR,),
        in_specs=[_rows(x.shape), _full(wl.shape), _full(wr.shape)],
        out_specs=[_rows((N, o)), _rows((N, o))],
        out_shape=[jax.ShapeDtypeStruct((N, o), jnp.float32)] * 2,
    )(x, wl, wr)


def _tc_mid(p, cnt, r, b, wl, wr, o_in, o_out):
    return pl.pallas_call(
        _tc_mid_body,
        grid=(N // _BR,),
        in_specs=[_part(o_in), _part(CW), _rows(r.shape), _full(b.shape),
                  _full(wl.shape), _full(wr.shape)],
        out_specs=[_rows((N, o_out)), _rows((N, o_out))],
        out_shape=[jax.ShapeDtypeStruct((N, o_out), jnp.float32)] * 2,
    )(p, cnt, r, b, wl, wr)


def _tc_dec(p, cnt, r, b, d0w, d0b, d1w, d1b, d2w, d2b):
    return pl.pallas_call(
        _tc_dec_body,
        grid=(N // _BR,),
        in_specs=[_part(32), _part(CW), _rows(r.shape), _full(b.shape),
                  _full(d0w.shape), _full(d0b.shape),
                  _full(d1w.shape), _full(d1b.shape),
                  _full(d2w.shape), _full(d2b.shape)],
        out_specs=_rows((N, D)),
        out_shape=jax.ShapeDtypeStruct((N, D), jnp.float32),
    )(p, cnt, r, b, d0w, d0b, d1w, d1b, d2w, d2b)


def kernel(x, edge_index, e0Wl, e0Wr, e0b, e1Wl, e1Wr, e1b, e2Wl, e2Wr, e2b,
           d0W, d0b, d1W, d1b, d2W, d2b):
    ei80 = edge_index.reshape(2, NW, EPW // 80, 80)
    ei100 = edge_index.reshape(2, NW, EPW // 100, 100)
    z128 = jnp.zeros((TROW + TAIL, 128), jnp.float32)
    zc = jnp.zeros((TROW + TAIL, CW), jnp.float32)

    t0, r0 = _tc_proj(x, e0Wl, e0Wr, 128)
    p0, cnt = _make_sc_aggregate(128, True, 80)(t0, ei80, z128, zc)
    t1, r1 = _tc_mid(p0, cnt, r0, e0b, e1Wl, e1Wr, 128, 64)
    (p1,) = _make_sc_aggregate(64, False, 100)(t1, ei100, z128[:, :64])
    t2, r2 = _tc_mid(p1, cnt, r1, e1b, e2Wl, e2Wr, 64, 32)
    (p2,) = _make_sc_aggregate(32, False, 100)(t2, ei100, z128[:, :32])
    return _tc_dec(p2, cnt, r2, e2b, d0W, d0b, d1W, d1b, d2W, d2b)


# speedup vs baseline: 2.9497x; 1.0351x over previous
import functools

import jax
import jax.numpy as jnp
from jax import lax
from jax.experimental import pallas as pl
from jax.experimental.pallas import tpu as pltpu
from jax.experimental.pallas import tpu_sc as plsc

N = 10000
E = 320000
D = 128

NC = 2
NS = 16
NW = NC * NS
EPW = E // NW
ACCR = N
TROW = 624
TAIL = N - NS * TROW
TAIL0 = NS * TROW
CW = 8


def _sc_body(with_count, o, k, nch, nbuf, *refs):
    if with_count:
        (table, ei, zeros_o, zeros_c, out, cnt_out,
         src_idx, dst_idx, rows0, rows1, ones_v, acc, cnt_acc,
         sem0, sem1, semc) = refs
    else:
        (table, ei, zeros_o, out, src_idx, dst_idx,
         rows0, rows1, rows2, rows3, acc, sem0, sem1, sem2, sem3) = refs
    c = lax.axis_index("c")
    s = lax.axis_index("s")
    wid = c * NS + s

    pltpu.sync_copy(ei.at[0, wid], src_idx)
    pltpu.sync_copy(ei.at[1, wid], dst_idx)

    row0 = s * TROW
    pltpu.sync_copy(zeros_o.at[pl.ds(0, TROW), :],
                    acc.at[pl.ds(row0, TROW), :])
    if with_count:
        pltpu.sync_copy(zeros_c.at[pl.ds(0, TROW), :],
                        cnt_acc.at[pl.ds(row0, TROW), :])

        def _onerow(i, _):
            ones_v[i, :] = jnp.ones((CW,), jnp.float32)
            return _
        lax.fori_loop(0, k, _onerow, None)

    @pl.when(s == 0)
    def _zero_tail():
        pltpu.sync_copy(zeros_o.at[pl.ds(TROW, TAIL), :],
                        acc.at[pl.ds(TAIL0, TAIL), :])
        if with_count:
            pltpu.sync_copy(zeros_c.at[pl.ds(TROW, TAIL), :],
                            cnt_acc.at[pl.ds(TAIL0, TAIL), :])
    plsc.subcore_barrier()

    def _consume(j, buf, sem):
        pltpu.make_async_copy(table.at[src_idx.at[j]], buf, sem).wait()
        pltpu.sync_copy(buf, acc.at[dst_idx.at[j]], add=True)
        if with_count:
            pltpu.async_copy(ones_v, cnt_acc.at[dst_idx.at[j]], semc,
                             add=True)

    def _gather(j, buf, sem):
        pltpu.async_copy(table.at[src_idx.at[j]], buf, sem)

    if nbuf == 2:
        def _chunk2(i, _):
            j0 = 2 * i
            _gather(j0 + 1, rows1, sem1)
            _consume(j0, rows0, sem0)
            _gather(j0 + 2, rows0, sem0)
            _consume(j0 + 1, rows1, sem1)
            return _

        _gather(0, rows0, sem0)
        lax.fori_loop(0, (nch - 1) // 2, _chunk2, None)
        _consume(nch - 1, rows0, sem0)
    else:
        def _chunk4(i, _):
            j0 = 4 * i
            _gather(j0 + 3, rows3, sem3)
            _consume(j0, rows0, sem0)
            _gather(j0 + 4, rows0, sem0)
            _consume(j0 + 1, rows1, sem1)
            _gather(j0 + 5, rows1, sem1)
            _consume(j0 + 2, rows2, sem2)
            _gather(j0 + 6, rows2, sem2)
            _consume(j0 + 3, rows3, sem3)
            return _

        _gather(0, rows0, sem0)
        _gather(1, rows1, sem1)
        _gather(2, rows2, sem2)
        lax.fori_loop(0, (nch - 4) // 4, _chunk4, None)
        _gather(nch - 1, rows3, sem3)
        _consume(nch - 4, rows0, sem0)
        _consume(nch - 3, rows1, sem1)
        _consume(nch - 2, rows2, sem2)
        _consume(nch - 1, rows3, sem3)
    if with_count:
        def _drain(i, _):
            pltpu.make_async_copy(ones_v, cnt_acc.at[dst_idx.at[0]],
                                  semc).wait()
            return _
        lax.fori_loop(0, nch, _drain, None)
    plsc.subcore_barrier()

    pltpu.sync_copy(acc.at[pl.ds(row0, TROW), :],
                    out.at[c, pl.ds(row0, TROW), :])
    if with_count:
        pltpu.sync_copy(cnt_acc.at[pl.ds(row0, TROW), :],
                        cnt_out.at[c, pl.ds(row0, TROW), :])

    @pl.when(s == 0)
    def _write_tail():
        pltpu.sync_copy(acc.at[pl.ds(TAIL0, TAIL), :],
                        out.at[c, pl.ds(TAIL0, TAIL), :])
        if with_count:
            pltpu.sync_copy(cnt_acc.at[pl.ds(TAIL0, TAIL), :],
                            cnt_out.at[c, pl.ds(TAIL0, TAIL), :])


@functools.lru_cache(maxsize=None)
def _make_sc_aggregate(o, with_count, k):
    nch = EPW // k
    nbuf = 2 if with_count else 4
    mesh = plsc.VectorSubcoreMesh(
        core_axis_name="c", subcore_axis_name="s",
        num_cores=NC, num_subcores=NS)
    out_type = [jax.ShapeDtypeStruct((NC, N, o), jnp.float32)]
    scratch = [
        pltpu.VMEM((nch, k), jnp.int32),
        pltpu.VMEM((nch, k), jnp.int32),
    ]
    scratch += [pltpu.VMEM((k, o), jnp.float32)] * nbuf
    if with_count:
        out_type.append(jax.ShapeDtypeStruct((NC, N, CW), jnp.float32))
        scratch.append(pltpu.VMEM((k, CW), jnp.float32))
    scratch.append(pltpu.VMEM_SHARED((ACCR, o), jnp.float32))
    if with_count:
        scratch.append(pltpu.VMEM_SHARED((ACCR, CW), jnp.float32))
    scratch.extend([pltpu.SemaphoreType.DMA] * (3 if with_count else 4))
    return pl.kernel(
        functools.partial(_sc_body, with_count, o, k, nch, nbuf),
        out_type=out_type,
        mesh=mesh,
        scratch_types=scratch,
        compiler_params=pltpu.CompilerParams(use_tc_tiling_on_sc=False),
        name=f"sc_seg_sum_{o}" + ("_cnt" if with_count else ""),
    )


_BR = 2000


def _dotT(a, w):
    return lax.dot_general(a, w, (((1,), (1,)), ((), ())),
                           preferred_element_type=jnp.float32)


def _tc_proj_body(x_ref, wl_ref, wr_ref, t_ref, r_ref):
    xb = x_ref[...]
    t_ref[...] = _dotT(xb, wl_ref[...])
    r_ref[...] = _dotT(xb, wr_ref[...])


def _combine(p_ref, cnt_ref, r_ref, b_ref):
    ssum = p_ref[0] + p_ref[1]
    cnts = cnt_ref[0, :, 0:1] + cnt_ref[1, :, 0:1]
    inv = 1.0 / jnp.maximum(cnts, 1.0)
    return jnp.maximum(ssum * inv + r_ref[...] + b_ref[...][None, :], 0.0)


def _tc_mid_body(p_ref, cnt_ref, r_ref, b_ref, wl_ref, wr_ref, t_ref, rr_ref):
    h = _combine(p_ref, cnt_ref, r_ref, b_ref)
    t_ref[...] = _dotT(h, wl_ref[...])
    rr_ref[...] = _dotT(h, wr_ref[...])


def _tc_dec_body(p_ref, cnt_ref, r_ref, b_ref,
                 d0w_ref, d0b_ref, d1w_ref, d1b_ref, d2w_ref, d2b_ref,
                 out_ref):
    h = _combine(p_ref, cnt_ref, r_ref, b_ref)
    h = jnp.maximum(_dotT(h, d0w_ref[...]) + d0b_ref[...][None, :], 0.0)
    h = jnp.maximum(_dotT(h, d1w_ref[...]) + d1b_ref[...][None, :], 0.0)
    out_ref[...] = jnp.maximum(
        _dotT(h, d2w_ref[...]) + d2b_ref[...][None, :], 0.0)


def _rows(shape):
    return pl.BlockSpec((_BR,) + shape[1:],
                        lambda i: (i,) + (0,) * (len(shape) - 1))


def _full(shape):
    return pl.BlockSpec(shape, lambda i: (0,) * len(shape))


def _part(o):
    return pl.BlockSpec((NC, _BR, o), lambda i: (0, i, 0))


def _tc_proj(x, wl, wr, o):
    return pl.pallas_call(
        _tc_proj_body,
        grid=(N // _BR,),
        in_specs=[_rows(x.shape), _full(wl.shape), _full(wr.shape)],
        out_specs=[_rows((N, o)), _rows((N, o))],
        out_shape=[jax.ShapeDtypeStruct((N, o), jnp.float32)] * 2,
    )(x, wl, wr)


def _tc_mid(p, cnt, r, b, wl, wr, o_in, o_out):
    return pl.pallas_call(
        _tc_mid_body,
        grid=(N // _BR,),
        in_specs=[_part(o_in), _part(CW), _rows(r.shape), _full(b.shape),
                  _full(wl.shape), _full(wr.shape)],
        out_specs=[_rows((N, o_out)), _rows((N, o_out))],
        out_shape=[jax.ShapeDtypeStruct((N, o_out), jnp.float32)] * 2,
    )(p, cnt, r, b, wl, wr)


def _tc_dec(p, cnt, r, b, d0w, d0b, d1w, d1b, d2w, d2b):
    return pl.pallas_call(
        _tc_dec_body,
        grid=(N // _BR,),
        in_specs=[_part(32), _part(CW), _rows(r.shape), _full(b.shape),
                  _full(d0w.shape), _full(d0b.shape),
                  _full(d1w.shape), _full(d1b.shape),
                  _full(d2w.shape), _full(d2b.shape)],
        out_specs=_rows((N, D)),
        out_shape=jax.ShapeDtypeStruct((N, D), jnp.float32),
    )(p, cnt, r, b, d0w, d0b, d1w, d1b, d2w, d2b)


def kernel(x, edge_index, e0Wl, e0Wr, e0b, e1Wl, e1Wr, e1b, e2Wl, e2Wr, e2b,
           d0W, d0b, d1W, d1b, d2W, d2b):
    ei80 = edge_index.reshape(2, NW, EPW // 80, 80)
    ei100 = edge_index.reshape(2, NW, EPW // 100, 100)
    z128 = jnp.zeros((TROW + TAIL, 128), jnp.float32)
    zc = jnp.zeros((TROW + TAIL, CW), jnp.float32)

    t0, r0 = _tc_proj(x, e0Wl, e0Wr, 128)
    p0, cnt = _make_sc_aggregate(128, True, 80)(t0, ei80, z128, zc)
    t1, r1 = _tc_mid(p0, cnt, r0, e0b, e1Wl, e1Wr, 128, 64)
    (p1,) = _make_sc_aggregate(64, False, 100)(t1, ei100, z128[:, :64])
    t2, r2 = _tc_mid(p1, cnt, r1, e1b, e2Wl, e2Wr, 64, 32)
    (p2,) = _make_sc_aggregate(32, False, 100)(t2, ei100, z128[:, :32])
    return _tc_dec(p2, cnt, r2, e2b, d0W, d0b, d1W, d1b, d2W, d2b)


# speedup vs baseline: 2.9913x; 1.0141x over previous
import functools

import jax
import jax.numpy as jnp
from jax import lax
from jax.experimental import pallas as pl
from jax.experimental.pallas import tpu as pltpu
from jax.experimental.pallas import tpu_sc as plsc

N = 10000
E = 320000
D = 128

NC = 2
NS = 16
NW = NC * NS
EPW = E // NW
ACCR = N
TROW = 624
TAIL = N - NS * TROW
TAIL0 = NS * TROW
CW = 8


def _sc_body(with_count, o, k, nch, nbuf, *refs):
    if with_count:
        (table, ei, zeros_o, zeros_c, out, cnt_out,
         src_idx, dst_idx, rows0, rows1, ones_v, acc, cnt_acc,
         sem0, sem1, semc) = refs
    else:
        (table, ei, zeros_o, out, src_idx, dst_idx,
         rows0, rows1, rows2, rows3, acc, sem0, sem1, sem2, sem3) = refs
    c = lax.axis_index("c")
    s = lax.axis_index("s")
    wid = c * NS + s

    pltpu.sync_copy(ei.at[0, wid], src_idx)
    pltpu.sync_copy(ei.at[1, wid], dst_idx)

    row0 = s * TROW
    pltpu.sync_copy(zeros_o.at[pl.ds(0, TROW), :],
                    acc.at[pl.ds(row0, TROW), :])
    if with_count:
        pltpu.sync_copy(zeros_c.at[pl.ds(0, TROW), :],
                        cnt_acc.at[pl.ds(row0, TROW), :])

        def _onerow(i, _):
            ones_v[i, :] = jnp.ones((CW,), jnp.float32)
            return _
        lax.fori_loop(0, k, _onerow, None)

    @pl.when(s == 0)
    def _zero_tail():
        pltpu.sync_copy(zeros_o.at[pl.ds(TROW, TAIL), :],
                        acc.at[pl.ds(TAIL0, TAIL), :])
        if with_count:
            pltpu.sync_copy(zeros_c.at[pl.ds(TROW, TAIL), :],
                            cnt_acc.at[pl.ds(TAIL0, TAIL), :])
    plsc.subcore_barrier()

    def _consume(j, buf, sem):
        pltpu.make_async_copy(table.at[src_idx.at[j]], buf, sem).wait()
        pltpu.sync_copy(buf, acc.at[dst_idx.at[j]], add=True)
        if with_count:
            pltpu.async_copy(ones_v, cnt_acc.at[dst_idx.at[j]], semc,
                             add=True)

    def _gather(j, buf, sem):
        pltpu.async_copy(table.at[src_idx.at[j]], buf, sem)

    if nbuf == 2:
        def _chunk2(i, _):
            j0 = 2 * i
            _gather(j0 + 1, rows1, sem1)
            _consume(j0, rows0, sem0)
            _gather(j0 + 2, rows0, sem0)
            _consume(j0 + 1, rows1, sem1)
            return _

        _gather(0, rows0, sem0)
        lax.fori_loop(0, (nch - 1) // 2, _chunk2, None)
        _consume(nch - 1, rows0, sem0)
    else:
        def _chunk4(i, _):
            j0 = 4 * i
            _gather(j0 + 3, rows3, sem3)
            _consume(j0, rows0, sem0)
            _gather(j0 + 4, rows0, sem0)
            _consume(j0 + 1, rows1, sem1)
            _gather(j0 + 5, rows1, sem1)
            _consume(j0 + 2, rows2, sem2)
            _gather(j0 + 6, rows2, sem2)
            _consume(j0 + 3, rows3, sem3)
            return _

        _gather(0, rows0, sem0)
        _gather(1, rows1, sem1)
        _gather(2, rows2, sem2)
        lax.fori_loop(0, (nch - 4) // 4, _chunk4, None)
        _gather(nch - 1, rows3, sem3)
        _consume(nch - 4, rows0, sem0)
        _consume(nch - 3, rows1, sem1)
        _consume(nch - 2, rows2, sem2)
        _consume(nch - 1, rows3, sem3)
    if with_count:
        def _drain(i, _):
            pltpu.make_async_copy(ones_v, cnt_acc.at[dst_idx.at[0]],
                                  semc).wait()
            return _
        lax.fori_loop(0, nch, _drain, None)
    plsc.subcore_barrier()

    pltpu.sync_copy(acc.at[pl.ds(row0, TROW), :],
                    out.at[c, pl.ds(row0, TROW), :])
    if with_count:
        pltpu.sync_copy(cnt_acc.at[pl.ds(row0, TROW), :],
                        cnt_out.at[c, pl.ds(row0, TROW), :])

    @pl.when(s == 0)
    def _write_tail():
        pltpu.sync_copy(acc.at[pl.ds(TAIL0, TAIL), :],
                        out.at[c, pl.ds(TAIL0, TAIL), :])
        if with_count:
            pltpu.sync_copy(cnt_acc.at[pl.ds(TAIL0, TAIL), :],
                            cnt_out.at[c, pl.ds(TAIL0, TAIL), :])


@functools.lru_cache(maxsize=None)
def _make_sc_aggregate(o, with_count, k):
    nch = EPW // k
    nbuf = 2 if with_count else 4
    mesh = plsc.VectorSubcoreMesh(
        core_axis_name="c", subcore_axis_name="s",
        num_cores=NC, num_subcores=NS)
    out_type = [jax.ShapeDtypeStruct((NC, N, o), jnp.float32)]
    scratch = [
        pltpu.VMEM((nch, k), jnp.int32),
        pltpu.VMEM((nch, k), jnp.int32),
    ]
    scratch += [pltpu.VMEM((k, o), jnp.float32)] * nbuf
    if with_count:
        out_type.append(jax.ShapeDtypeStruct((NC, N, CW), jnp.float32))
        scratch.append(pltpu.VMEM((k, CW), jnp.float32))
    scratch.append(pltpu.VMEM_SHARED((ACCR, o), jnp.float32))
    if with_count:
        scratch.append(pltpu.VMEM_SHARED((ACCR, CW), jnp.float32))
    scratch.extend([pltpu.SemaphoreType.DMA] * (3 if with_count else 4))
    return pl.kernel(
        functools.partial(_sc_body, with_count, o, k, nch, nbuf),
        out_type=out_type,
        mesh=mesh,
        scratch_types=scratch,
        compiler_params=pltpu.CompilerParams(use_tc_tiling_on_sc=False),
        name=f"sc_seg_sum_{o}" + ("_cnt" if with_count else ""),
    )


_BR = 2000


def _dotT(a, w):
    return lax.dot_general(a, w, (((1,), (1,)), ((), ())),
                           preferred_element_type=jnp.float32)


def _tc_proj_body(x_ref, wl_ref, wr_ref, t_ref, r_ref):
    xb = x_ref[...]
    t_ref[...] = _dotT(xb, wl_ref[...])
    r_ref[...] = _dotT(xb, wr_ref[...])


def _combine(p_ref, cnt_ref, r_ref, b_ref):
    ssum = p_ref[0] + p_ref[1]
    cnts = cnt_ref[0, :, 0:1] + cnt_ref[1, :, 0:1]
    inv = 1.0 / jnp.maximum(cnts, 1.0)
    return jnp.maximum(ssum * inv + r_ref[...] + b_ref[...][None, :], 0.0)


def _tc_mid_body(p_ref, cnt_ref, r_ref, b_ref, wl_ref, wr_ref, t_ref, rr_ref):
    h = _combine(p_ref, cnt_ref, r_ref, b_ref)
    t_ref[...] = _dotT(h, wl_ref[...])
    rr_ref[...] = _dotT(h, wr_ref[...])


def _tc_dec_body(p_ref, cnt_ref, r_ref, b_ref,
                 d0w_ref, d0b_ref, d1w_ref, d1b_ref, d2w_ref, d2b_ref,
                 out_ref):
    h = _combine(p_ref, cnt_ref, r_ref, b_ref)
    h = jnp.maximum(_dotT(h, d0w_ref[...]) + d0b_ref[...][None, :], 0.0)
    h = jnp.maximum(_dotT(h, d1w_ref[...]) + d1b_ref[...][None, :], 0.0)
    out_ref[...] = jnp.maximum(
        _dotT(h, d2w_ref[...]) + d2b_ref[...][None, :], 0.0)


def _rows(shape):
    return pl.BlockSpec((_BR,) + shape[1:],
                        lambda i: (i,) + (0,) * (len(shape) - 1))


def _full(shape):
    return pl.BlockSpec(shape, lambda i: (0,) * len(shape))


def _part(o):
    return pl.BlockSpec((NC, _BR, o), lambda i: (0, i, 0))


def _tc_proj(x, wl, wr, o):
    return pl.pallas_call(
        _tc_proj_body,
        grid=(N // _BR,),
        in_specs=[_rows(x.shape), _full(wl.shape), _full(wr.shape)],
        out_specs=[_rows((N, o)), _rows((N, o))],
        out_shape=[jax.ShapeDtypeStruct((N, o), jnp.float32)] * 2,
    )(x, wl, wr)


def _tc_mid(p, cnt, r, b, wl, wr, o_in, o_out):
    return pl.pallas_call(
        _tc_mid_body,
        grid=(N // _BR,),
        in_specs=[_part(o_in), _part(CW), _rows(r.shape), _full(b.shape),
                  _full(wl.shape), _full(wr.shape)],
        out_specs=[_rows((N, o_out)), _rows((N, o_out))],
        out_shape=[jax.ShapeDtypeStruct((N, o_out), jnp.float32)] * 2,
    )(p, cnt, r, b, wl, wr)


def _tc_dec(p, cnt, r, b, d0w, d0b, d1w, d1b, d2w, d2b):
    return pl.pallas_call(
        _tc_dec_body,
        grid=(N // _BR,),
        in_specs=[_part(32), _part(CW), _rows(r.shape), _full(b.shape),
                  _full(d0w.shape), _full(d0b.shape),
                  _full(d1w.shape), _full(d1b.shape),
                  _full(d2w.shape), _full(d2b.shape)],
        out_specs=_rows((N, D)),
        out_shape=jax.ShapeDtypeStruct((N, D), jnp.float32),
    )(p, cnt, r, b, d0w, d0b, d1w, d1b, d2w, d2b)


def kernel(x, edge_index, e0Wl, e0Wr, e0b, e1Wl, e1Wr, e1b, e2Wl, e2Wr, e2b,
           d0W, d0b, d1W, d1b, d2W, d2b):
    ei80 = edge_index.reshape(2, NW, EPW // 80, 80)
    ei125 = edge_index.reshape(2, NW, EPW // 125, 125)
    z128 = jnp.zeros((TROW + TAIL, 128), jnp.float32)
    zc = jnp.zeros((TROW + TAIL, CW), jnp.float32)

    t0, r0 = _tc_proj(x, e0Wl, e0Wr, 128)
    p0, cnt = _make_sc_aggregate(128, True, 80)(t0, ei80, z128, zc)
    t1, r1 = _tc_mid(p0, cnt, r0, e0b, e1Wl, e1Wr, 128, 64)
    (p1,) = _make_sc_aggregate(64, False, 125)(t1, ei125, z128[:, :64])
    t2, r2 = _tc_mid(p1, cnt, r1, e1b, e2Wl, e2Wr, 64, 32)
    (p2,) = _make_sc_aggregate(32, False, 125)(t2, ei125, z128[:, :32])
    return _tc_dec(p2, cnt, r2, e2b, d0W, d0b, d1W, d1b, d2W, d2b)
